# Initial kernel scaffold; baseline (speedup 1.0000x reference)
import jax
import jax.numpy as jnp
from jax.experimental import pallas as pl


def kernel(x23, indices, emb_table, gamma, beta):
    raise NotImplementedError("write your pallas kernel here")


# speedup vs baseline: 1.1758x; 1.1758x over previous
import functools

import jax
import jax.numpy as jnp
from jax import lax
from jax.experimental import pallas as pl
from jax.experimental.pallas import tpu as pltpu
from jax.experimental.pallas import tpu_sc as plsc

_B = 384
_D = 128
_ROWS_PER = 16
_NWORK = _B // _ROWS_PER


def _sc_gather(table, idx):
    mesh = plsc.VectorSubcoreMesh(core_axis_name="c", subcore_axis_name="s")
    nc = plsc.get_sparse_core_info().num_cores

    @functools.partial(
        pl.kernel,
        mesh=mesh,
        out_type=jax.ShapeDtypeStruct((_B, _D), jnp.float32),
        scratch_types=[
            pltpu.VMEM((_ROWS_PER,), jnp.int32),
            pltpu.VMEM((_ROWS_PER, _D), jnp.float32),
            pltpu.SemaphoreType.DMA,
        ],
    )
    def k(table_hbm, idx_hbm, out_hbm, idx_v, rows_v, sem):
        wid = lax.axis_index("s") * nc + lax.axis_index("c")

        @pl.when(wid < _NWORK)
        def _():
            base = wid * _ROWS_PER
            pltpu.sync_copy(idx_hbm.at[pl.ds(base, _ROWS_PER)], idx_v)
            pltpu.async_copy(table_hbm.at[idx_v], rows_v, sem).wait()
            pltpu.sync_copy(rows_v, out_hbm.at[pl.ds(base, _ROWS_PER)])

    return k(table, idx)


def _ln_body(x_ref, e_ref, g_ref, b_ref, o_ref):
    x = x_ref[...]
    e = e_ref[...][None, :, :]
    y = x + e
    mean = jnp.mean(y, axis=-1, keepdims=True)
    c = y - mean
    var = jnp.mean(c * c, axis=-1, keepdims=True)
    inv = lax.rsqrt(var + 1e-12)
    g = g_ref[...][None, :, :]
    b = b_ref[...][None, :, :]
    o_ref[...] = c * inv * g + b


def _tc_add_ln(x23, emb_rows, gamma, beta):
    bsz = x23.shape[0]
    bb = 8
    grid = (bsz // bb,)
    return pl.pallas_call(
        _ln_body,
        grid=grid,
        in_specs=[
            pl.BlockSpec((bb, _B, _D), lambda i: (i, 0, 0)),
            pl.BlockSpec((_B, _D), lambda i: (0, 0)),
            pl.BlockSpec((1, _D), lambda i: (0, 0)),
            pl.BlockSpec((1, _D), lambda i: (0, 0)),
        ],
        out_specs=pl.BlockSpec((bb, _B, _D), lambda i: (i, 0, 0)),
        out_shape=jax.ShapeDtypeStruct(x23.shape, jnp.float32),
        compiler_params=pltpu.CompilerParams(
            dimension_semantics=("arbitrary",),
        ),
    )(x23, emb_rows, gamma.reshape(1, _D), beta.reshape(1, _D))


def kernel(x23, indices, emb_table, gamma, beta):
    idx = jnp.asarray(indices).reshape(-1).astype(jnp.int32)
    emb_rows = _sc_gather(emb_table, idx)
    return _tc_add_ln(x23, emb_rows, gamma, beta)


# speedup vs baseline: 1.3405x; 1.1401x over previous
import functools

import jax
import jax.numpy as jnp
from jax import lax
from jax.experimental import pallas as pl
from jax.experimental.pallas import tpu as pltpu
from jax.experimental.pallas import tpu_sc as plsc

_B = 384
_D = 128
_ROWS_PER = 16
_NWORK = _B // _ROWS_PER


def _sc_gather(table, idx):
    mesh = plsc.VectorSubcoreMesh(core_axis_name="c", subcore_axis_name="s")
    nc = plsc.get_sparse_core_info().num_cores

    @functools.partial(
        pl.kernel,
        mesh=mesh,
        out_type=jax.ShapeDtypeStruct((_B, _D), jnp.float32),
        scratch_types=[
            pltpu.VMEM((_ROWS_PER,), jnp.int32),
            pltpu.VMEM((_ROWS_PER, _D), jnp.float32),
            pltpu.SemaphoreType.DMA,
        ],
    )
    def k(table_hbm, idx_hbm, out_hbm, idx_v, rows_v, sem):
        wid = lax.axis_index("s") * nc + lax.axis_index("c")

        @pl.when(wid < _NWORK)
        def _():
            base = wid * _ROWS_PER
            pltpu.sync_copy(idx_hbm.at[pl.ds(base, _ROWS_PER)], idx_v)
            pltpu.async_copy(table_hbm.at[idx_v], rows_v, sem).wait()
            pltpu.sync_copy(rows_v, out_hbm.at[pl.ds(base, _ROWS_PER)])

    return k(table, idx)


def _ln_body(x_ref, e_ref, g_ref, b_ref, o_ref):
    x = x_ref[...]
    e = e_ref[...][None, :, :]
    y = x + e
    mean = jnp.mean(y, axis=-1, keepdims=True)
    c = y - mean
    var = jnp.mean(c * c, axis=-1, keepdims=True)
    inv = lax.rsqrt(var + 1e-12)
    g = g_ref[...][None, :, :]
    b = b_ref[...][None, :, :]
    o_ref[...] = c * inv * g + b


def _tc_add_ln(x23, emb_rows, gamma, beta):
    bsz = x23.shape[0]
    bb = 16
    grid = (bsz // bb,)
    return pl.pallas_call(
        _ln_body,
        grid=grid,
        in_specs=[
            pl.BlockSpec((bb, _B, _D), lambda i: (i, 0, 0)),
            pl.BlockSpec((_B, _D), lambda i: (0, 0)),
            pl.BlockSpec((1, _D), lambda i: (0, 0)),
            pl.BlockSpec((1, _D), lambda i: (0, 0)),
        ],
        out_specs=pl.BlockSpec((bb, _B, _D), lambda i: (i, 0, 0)),
        out_shape=jax.ShapeDtypeStruct(x23.shape, jnp.float32),
        compiler_params=pltpu.CompilerParams(
            dimension_semantics=("arbitrary",),
        ),
    )(x23, emb_rows, gamma.reshape(1, _D), beta.reshape(1, _D))


def kernel(x23, indices, emb_table, gamma, beta):
    idx = jnp.asarray(indices).reshape(-1).astype(jnp.int32)
    emb_rows = _sc_gather(emb_table, idx)
    return _tc_add_ln(x23, emb_rows, gamma, beta)


# speedup vs baseline: 1.4196x; 1.0590x over previous
import functools

import jax
import jax.numpy as jnp
from jax import lax
from jax.experimental import pallas as pl
from jax.experimental.pallas import tpu as pltpu
from jax.experimental.pallas import tpu_sc as plsc

_B = 384
_D = 128
_ROWS_PER = 16
_NWORK = _B // _ROWS_PER


def _sc_gather(table, idx):
    mesh = plsc.VectorSubcoreMesh(core_axis_name="c", subcore_axis_name="s")
    nc = plsc.get_sparse_core_info().num_cores

    @functools.partial(
        pl.kernel,
        mesh=mesh,
        out_type=jax.ShapeDtypeStruct((_B, _D), jnp.float32),
        scratch_types=[
            pltpu.VMEM((_ROWS_PER,), jnp.int32),
            pltpu.VMEM((_ROWS_PER, _D), jnp.float32),
            pltpu.SemaphoreType.DMA,
        ],
    )
    def k(table_hbm, idx_hbm, out_hbm, idx_v, rows_v, sem):
        wid = lax.axis_index("s") * nc + lax.axis_index("c")

        @pl.when(wid < _NWORK)
        def _():
            base = wid * _ROWS_PER
            pltpu.sync_copy(idx_hbm.at[pl.ds(base, _ROWS_PER)], idx_v)
            pltpu.async_copy(table_hbm.at[idx_v], rows_v, sem).wait()
            pltpu.sync_copy(rows_v, out_hbm.at[pl.ds(base, _ROWS_PER)])

    return k(table, idx)


def _ln_body(x_ref, e_ref, g_ref, b_ref, o_ref):
    x = x_ref[...]
    e = e_ref[...][None, :, :]
    y = x + e
    mean = jnp.mean(y, axis=-1, keepdims=True)
    c = y - mean
    var = jnp.mean(c * c, axis=-1, keepdims=True)
    inv = lax.rsqrt(var + 1e-12)
    g = g_ref[...][None, :, :]
    b = b_ref[...][None, :, :]
    o_ref[...] = c * inv * g + b


def _tc_add_ln(x23, emb_rows, gamma, beta):
    bsz = x23.shape[0]
    bb = 32
    grid = (bsz // bb,)
    return pl.pallas_call(
        _ln_body,
        grid=grid,
        in_specs=[
            pl.BlockSpec((bb, _B, _D), lambda i: (i, 0, 0)),
            pl.BlockSpec((_B, _D), lambda i: (0, 0)),
            pl.BlockSpec((1, _D), lambda i: (0, 0)),
            pl.BlockSpec((1, _D), lambda i: (0, 0)),
        ],
        out_specs=pl.BlockSpec((bb, _B, _D), lambda i: (i, 0, 0)),
        out_shape=jax.ShapeDtypeStruct(x23.shape, jnp.float32),
        compiler_params=pltpu.CompilerParams(
            dimension_semantics=("arbitrary",),
        ),
    )(x23, emb_rows, gamma.reshape(1, _D), beta.reshape(1, _D))


def kernel(x23, indices, emb_table, gamma, beta):
    idx = jnp.asarray(indices).reshape(-1).astype(jnp.int32)
    emb_rows = _sc_gather(emb_table, idx)
    return _tc_add_ln(x23, emb_rows, gamma, beta)
